# Initial kernel scaffold; baseline (speedup 1.0000x reference)
import functools

import jax
import jax.numpy as jnp
from jax import lax
from jax.experimental import pallas as pl
from jax.experimental.pallas import tpu as pltpu
from jax.experimental.pallas import tpu_sc as plsc

NUM_EMBEDDINGS = 1000000
EMBEDDING_DIM = 32
BATCH = 16384
FIELDS = 26

_TOTAL = BATCH * FIELDS
_NW = 32
_PER_W = _TOTAL // _NW
_CHUNK = 1024
_NCHUNK = _PER_W // _CHUNK

assert _PER_W % _CHUNK == 0


def _make_kernel():
    mesh = plsc.VectorSubcoreMesh(core_axis_name="c", subcore_axis_name="s")

    @functools.partial(
        pl.kernel,
        mesh=mesh,
        out_type=jax.ShapeDtypeStruct((_TOTAL, EMBEDDING_DIM), jnp.float32),
        scratch_types=[
            pltpu.VMEM((_CHUNK,), jnp.int32),
            pltpu.VMEM((_CHUNK, EMBEDDING_DIM), jnp.float32),
            pltpu.SemaphoreType.DMA,
        ],
    )
    def emb_kernel(table_hbm, idx_hbm, out_hbm, idx_v, rows_v, sem):
        wid = lax.axis_index("s") * 2 + lax.axis_index("c")
        base = wid * _PER_W
        for g in range(_NCHUNK):
            off = base + g * _CHUNK
            pltpu.sync_copy(idx_hbm.at[pl.ds(off, _CHUNK)], idx_v)
            pltpu.async_copy(table_hbm.at[idx_v], rows_v, sem).wait()
            pltpu.sync_copy(rows_v, out_hbm.at[pl.ds(off, _CHUNK)])

    return emb_kernel


_EMB = _make_kernel()


@jax.jit
def kernel(x, weight):
    idx = x.reshape(-1).astype(jnp.int32)
    out = _EMB(weight, idx)
    return out.reshape(BATCH, FIELDS, EMBEDDING_DIM)


# speedup vs baseline: 1.5468x; 1.5468x over previous
import functools

import jax
import jax.numpy as jnp
from jax import lax
from jax.experimental import pallas as pl
from jax.experimental.pallas import tpu as pltpu
from jax.experimental.pallas import tpu_sc as plsc

NUM_EMBEDDINGS = 1000000
EMBEDDING_DIM = 32
BATCH = 16384
FIELDS = 26

_TOTAL = BATCH * FIELDS
_NW = 32
_PER_W = _TOTAL // _NW
_CHUNK = 1024
_NCHUNK = _PER_W // _CHUNK

assert _PER_W % _CHUNK == 0


def _make_kernel():
    mesh = plsc.VectorSubcoreMesh(core_axis_name="c", subcore_axis_name="s")

    @functools.partial(
        pl.kernel,
        mesh=mesh,
        out_type=jax.ShapeDtypeStruct((_TOTAL, EMBEDDING_DIM), jnp.float32),
        scratch_types=[
            pltpu.VMEM((_CHUNK,), jnp.int32),
            pltpu.VMEM((_CHUNK, EMBEDDING_DIM), jnp.float32),
            pltpu.SemaphoreType.DMA,
        ],
        compiler_params=pltpu.CompilerParams(use_tc_tiling_on_sc=False),
    )
    def emb_kernel(table_hbm, idx_hbm, out_hbm, idx_v, rows_v, sem):
        wid = lax.axis_index("s") * 2 + lax.axis_index("c")
        base = wid * _PER_W
        for g in range(_NCHUNK):
            off = base + g * _CHUNK
            pltpu.sync_copy(idx_hbm.at[pl.ds(off, _CHUNK)], idx_v)
            pltpu.async_copy(table_hbm.at[idx_v], rows_v, sem).wait()
            pltpu.sync_copy(rows_v, out_hbm.at[pl.ds(off, _CHUNK)])

    return emb_kernel


_EMB = _make_kernel()


@jax.jit
def kernel(x, weight):
    idx = x.reshape(-1).astype(jnp.int32)
    out = _EMB(weight, idx)
    return out.reshape(BATCH, FIELDS, EMBEDDING_DIM)


# speedup vs baseline: 1.5758x; 1.0187x over previous
import functools

import jax
import jax.numpy as jnp
from jax import lax
from jax.experimental import pallas as pl
from jax.experimental.pallas import tpu as pltpu
from jax.experimental.pallas import tpu_sc as plsc

NUM_EMBEDDINGS = 1000000
EMBEDDING_DIM = 32
BATCH = 16384
FIELDS = 26

_TOTAL = BATCH * FIELDS
_NW = 32
_PER_W = _TOTAL // _NW
_CHUNK = 1664
_NCHUNK = _PER_W // _CHUNK
_NBUF = 2

assert _PER_W % _CHUNK == 0 and _CHUNK % 8 == 0


def _make_kernel():
    mesh = plsc.VectorSubcoreMesh(core_axis_name="c", subcore_axis_name="s")

    @functools.partial(
        pl.kernel,
        mesh=mesh,
        out_type=jax.ShapeDtypeStruct((_TOTAL, EMBEDDING_DIM), jnp.float32),
        scratch_types=[
            pltpu.VMEM((_PER_W,), jnp.int32),
            pltpu.VMEM((_NBUF, _CHUNK, EMBEDDING_DIM), jnp.float32),
            [pltpu.SemaphoreType.DMA] * _NBUF,
            [pltpu.SemaphoreType.DMA] * _NBUF,
        ],
        compiler_params=pltpu.CompilerParams(use_tc_tiling_on_sc=False),
    )
    def emb_kernel(table_hbm, idx_hbm, out_hbm, idx_v, rows_v, gsems, ssems):
        wid = lax.axis_index("s") * 2 + lax.axis_index("c")
        base = wid * _PER_W
        pltpu.sync_copy(idx_hbm.at[pl.ds(base, _PER_W)], idx_v)

        def start_gather(g, b):
            return pltpu.async_copy(
                table_hbm.at[idx_v.at[pl.ds(g * _CHUNK, _CHUNK)]],
                rows_v.at[b],
                gsems[b],
            )

        def start_store(g, b):
            return pltpu.async_copy(
                rows_v.at[b],
                out_hbm.at[pl.ds(base + g * _CHUNK, _CHUNK)],
                ssems[b],
            )

        gathers = {}
        stores = {}
        gathers[0] = start_gather(0, 0)
        for g in range(_NCHUNK):
            b = g % _NBUF
            nb = (g + 1) % _NBUF
            if g + 1 < _NCHUNK:
                prev = g + 1 - _NBUF
                if prev >= 0:
                    stores.pop(prev).wait()
                gathers[g + 1] = start_gather(g + 1, nb)
            gathers.pop(g).wait()
            stores[g] = start_store(g, b)
        for g in sorted(stores):
            stores.pop(g).wait()

    return emb_kernel


_EMB = _make_kernel()


@jax.jit
def kernel(x, weight):
    idx = x.reshape(-1).astype(jnp.int32)
    out = _EMB(weight, idx)
    return out.reshape(BATCH, FIELDS, EMBEDDING_DIM)
